# baseline (speedup 1.0000x reference)
import functools

import jax
import jax.numpy as jnp
from jax import lax
from jax.experimental import pallas as pl
from jax.experimental.pallas import tpu as pltpu
from jax.experimental.pallas import tpu_sc as plsc

NUM_EXPERTS = 8
HIDDEN = 1024
INTER = 2048
T = 4096
TCHUNK = 512
RCHUNKS = T // TCHUNK
BLK = 128
P = T + NUM_EXPERTS * BLK
NB = P // BLK
NC, NS = 2, 16
NW = NC * NS
TPW = T // NW
HALF = TPW // 2


def _router_body(hs_ref, rw_ref, xs_ref, idx_ref, rank_ref, counts_ref,
                 cnt_scr):
    t = pl.program_id(0)

    @pl.when(t == 0)
    def _():
        cnt_scr[...] = jnp.zeros((1, 128), jnp.float32)

    x32 = hs_ref[...]
    logits = jnp.dot(x32, rw_ref[...], preferred_element_type=jnp.float32)
    m = jnp.max(logits, axis=1, keepdims=True)
    s = jax.nn.sigmoid(m)
    iota_e = lax.broadcasted_iota(jnp.int32, logits.shape, 1)
    idx = jnp.min(jnp.where(logits == m, iota_e, NUM_EXPERTS), axis=1,
                  keepdims=True)
    onehot = (iota_e == idx).astype(jnp.float32)
    ri = lax.broadcasted_iota(jnp.int32, (TCHUNK, TCHUNK), 0)
    ci = lax.broadcasted_iota(jnp.int32, (TCHUNK, TCHUNK), 1)
    tril = (ci < ri).astype(jnp.float32)
    prev = jnp.dot(tril, onehot, preferred_element_type=jnp.float32)
    rank_local = jnp.sum(prev * onehot, axis=1, keepdims=True)
    cnt = cnt_scr[:, :NUM_EXPERTS]
    carry = jnp.sum(jnp.broadcast_to(cnt, onehot.shape) * onehot, axis=1,
                    keepdims=True)
    idx_ref[...] = idx
    rank_ref[...] = (rank_local + carry).astype(jnp.int32)
    xs_ref[...] = (x32 * s).astype(jnp.bfloat16)
    cnt_scr[:, :NUM_EXPERTS] = cnt + jnp.sum(onehot, axis=0, keepdims=True)

    @pl.when(t == RCHUNKS - 1)
    def _():
        counts_ref[...] = cnt_scr[...]


def _run_router(hs, rw_t):
    return pl.pallas_call(
        _router_body,
        grid=(RCHUNKS,),
        in_specs=[
            pl.BlockSpec((TCHUNK, HIDDEN), lambda t: (t, 0)),
            pl.BlockSpec((HIDDEN, NUM_EXPERTS), lambda t: (0, 0)),
        ],
        out_specs=[
            pl.BlockSpec((TCHUNK, HIDDEN), lambda t: (t, 0)),
            pl.BlockSpec((TCHUNK, 1), lambda t: (t, 0)),
            pl.BlockSpec((TCHUNK, 1), lambda t: (t, 0)),
            pl.BlockSpec((1, 128), lambda t: (0, 0)),
        ],
        out_shape=[
            jax.ShapeDtypeStruct((T, HIDDEN), jnp.bfloat16),
            jax.ShapeDtypeStruct((T, 1), jnp.int32),
            jax.ShapeDtypeStruct((T, 1), jnp.int32),
            jax.ShapeDtypeStruct((1, 128), jnp.float32),
        ],
        scratch_shapes=[pltpu.VMEM((1, 128), jnp.float32)],
    )(hs, rw_t)


def _slot_body(idx_ref, rank_ref, counts_ref, slot_ref):
    idx = idx_ref[...]
    cnt = counts_ref[:, :NUM_EXPERTS]
    g_pad = jnp.ceil(cnt / BLK) * BLK
    r8 = lax.broadcasted_iota(jnp.int32, (NUM_EXPERTS, NUM_EXPERTS), 0)
    c8 = lax.broadcasted_iota(jnp.int32, (NUM_EXPERTS, NUM_EXPERTS), 1)
    strict_upper = (r8 < c8).astype(jnp.float32)
    base = jnp.dot(g_pad, strict_upper, preferred_element_type=jnp.float32)
    iota_e = lax.broadcasted_iota(jnp.int32, (T, NUM_EXPERTS), 1)
    oh = (iota_e == idx).astype(jnp.float32)
    basetok = jnp.sum(jnp.broadcast_to(base, oh.shape) * oh, axis=1,
                      keepdims=True)
    slot_ref[...] = rank_ref[...] + basetok.astype(jnp.int32)


def _run_slot(idx2, rank2, counts_row):
    return pl.pallas_call(
        _slot_body,
        grid=(1,),
        in_specs=[
            pl.BlockSpec((T, 1), lambda i: (0, 0)),
            pl.BlockSpec((T, 1), lambda i: (0, 0)),
            pl.BlockSpec((1, 128), lambda i: (0, 0)),
        ],
        out_specs=pl.BlockSpec((T, 1), lambda i: (0, 0)),
        out_shape=jax.ShapeDtypeStruct((T, 1), jnp.int32),
    )(idx2, rank2, counts_row)


@functools.lru_cache(maxsize=None)
def _sc_kernels():
    mesh = plsc.VectorSubcoreMesh(core_axis_name="c", subcore_axis_name="s",
                                  num_cores=NC, num_subcores=NS)

    @functools.partial(
        pl.kernel,
        out_type=jax.ShapeDtypeStruct((P, HIDDEN // 2), jnp.int32),
        mesh=mesh,
        scratch_types=[
            pltpu.VMEM((HALF,), jnp.int32),
            pltpu.VMEM((HALF,), jnp.int32),
            pltpu.VMEM((HALF, HIDDEN // 2), jnp.int32),
            pltpu.VMEM((HALF, HIDDEN // 2), jnp.int32),
            pltpu.SemaphoreType.DMA,
            pltpu.SemaphoreType.DMA,
            pltpu.SemaphoreType.DMA,
        ],
    )
    def _scatter_kernel(xs_hbm, slot_hbm, xsorted_hbm, slot_lo, slot_hi,
                        rows0, rows1, sem0, sem1, sem_s):
        wid = lax.axis_index("s") * NC + lax.axis_index("c")
        t0 = wid * TPW
        pltpu.sync_copy(slot_hbm.at[pl.ds(t0, HALF)], slot_lo)
        pltpu.sync_copy(slot_hbm.at[pl.ds(t0 + HALF, HALF)], slot_hi)
        c0 = pltpu.async_copy(xs_hbm.at[pl.ds(t0, HALF)], rows0, sem0)
        c1 = pltpu.async_copy(xs_hbm.at[pl.ds(t0 + HALF, HALF)], rows1, sem1)
        c0.wait()
        s0 = pltpu.async_copy(rows0, xsorted_hbm.at[slot_lo], sem_s)
        c1.wait()
        s1 = pltpu.async_copy(rows1, xsorted_hbm.at[slot_hi], sem_s)
        s0.wait()
        s1.wait()

    @functools.partial(
        pl.kernel,
        out_type=jax.ShapeDtypeStruct((T, HIDDEN // 2), jnp.int32),
        mesh=mesh,
        scratch_types=[
            pltpu.VMEM((HALF,), jnp.int32),
            pltpu.VMEM((HALF,), jnp.int32),
            pltpu.VMEM((HALF, HIDDEN // 2), jnp.int32),
            pltpu.VMEM((HALF, HIDDEN // 2), jnp.int32),
            pltpu.SemaphoreType.DMA,
            pltpu.SemaphoreType.DMA,
        ],
    )
    def _gather_kernel(ys_hbm, slot_hbm, out_hbm, slot_lo, slot_hi, rows0,
                       rows1, sem0, sem1):
        wid = lax.axis_index("s") * NC + lax.axis_index("c")
        t0 = wid * TPW
        pltpu.sync_copy(slot_hbm.at[pl.ds(t0, HALF)], slot_lo)
        pltpu.sync_copy(slot_hbm.at[pl.ds(t0 + HALF, HALF)], slot_hi)
        g0 = pltpu.async_copy(ys_hbm.at[slot_lo], rows0, sem0)
        g1 = pltpu.async_copy(ys_hbm.at[slot_hi], rows1, sem1)
        g0.wait()
        pltpu.sync_copy(rows0, out_hbm.at[pl.ds(t0, HALF)])
        g1.wait()
        pltpu.sync_copy(rows1, out_hbm.at[pl.ds(t0 + HALF, HALF)])

    return _scatter_kernel, _gather_kernel


def _ffn_body(be_ref, xs_ref, gu_ref, dn_ref, out_ref):
    x = xs_ref[...]
    gu = jnp.dot(x, gu_ref[0], preferred_element_type=jnp.float32)
    gate = gu[:, :INTER]
    up = gu[:, INTER:]
    gated = (up * (gate * jax.nn.sigmoid(gate))).astype(jnp.bfloat16)
    out_ref[...] = jnp.dot(gated, dn_ref[0],
                           preferred_element_type=jnp.float32
                           ).astype(jnp.bfloat16)


def _run_ffn(bexp, xsorted, gu_bf, dn_bf):
    grid_spec = pltpu.PrefetchScalarGridSpec(
        num_scalar_prefetch=1,
        grid=(NB,),
        in_specs=[
            pl.BlockSpec((BLK, HIDDEN), lambda j, be: (j, 0)),
            pl.BlockSpec((1, HIDDEN, 2 * INTER), lambda j, be: (be[j], 0, 0)),
            pl.BlockSpec((1, INTER, HIDDEN), lambda j, be: (be[j], 0, 0)),
        ],
        out_specs=pl.BlockSpec((BLK, HIDDEN), lambda j, be: (j, 0)),
    )
    return pl.pallas_call(
        _ffn_body,
        grid_spec=grid_spec,
        out_shape=jax.ShapeDtypeStruct((P, HIDDEN), jnp.bfloat16),
    )(bexp, xsorted, gu_bf, dn_bf)


def _shared_body(hs_ref, yt_ref, sg_ref, su_ref, sd_ref, out_ref):
    x = hs_ref[...].astype(jnp.bfloat16)
    g = jnp.dot(x, sg_ref[...], preferred_element_type=jnp.float32)
    u = jnp.dot(x, su_ref[...], preferred_element_type=jnp.float32)
    act = (u * (g * jax.nn.sigmoid(g))).astype(jnp.bfloat16)
    sh = jnp.dot(act, sd_ref[...], preferred_element_type=jnp.float32)
    out_ref[...] = sh + yt_ref[...].astype(jnp.float32)


def _run_shared(hs, ys_tok, sg_t, su_t, sd_t):
    return pl.pallas_call(
        _shared_body,
        grid=(RCHUNKS,),
        in_specs=[
            pl.BlockSpec((TCHUNK, HIDDEN), lambda t: (t, 0)),
            pl.BlockSpec((TCHUNK, HIDDEN), lambda t: (t, 0)),
            pl.BlockSpec((HIDDEN, INTER), lambda t: (0, 0)),
            pl.BlockSpec((HIDDEN, INTER), lambda t: (0, 0)),
            pl.BlockSpec((INTER, HIDDEN), lambda t: (0, 0)),
        ],
        out_specs=pl.BlockSpec((TCHUNK, HIDDEN), lambda t: (t, 0)),
        out_shape=jax.ShapeDtypeStruct((T, HIDDEN), jnp.float32),
    )(hs, ys_tok, sg_t, su_t, sd_t)


def kernel(hidden_states, gate_up_proj, down_proj, router_w, shared_gate_w,
           shared_up_w, shared_down_w):
    B, S, H = hidden_states.shape
    hs = hidden_states.reshape(B * S, H)
    rw_t = router_w.T.astype(jnp.float32)
    gu_bf = gate_up_proj.astype(jnp.bfloat16)
    dn_bf = down_proj.astype(jnp.bfloat16)
    sg_t = shared_gate_w.T.astype(jnp.bfloat16)
    su_t = shared_up_w.T.astype(jnp.bfloat16)
    sd_t = shared_down_w.T.astype(jnp.bfloat16)

    xs, idx2, rank2, counts_row = _run_router(hs, rw_t)
    slot = _run_slot(idx2, rank2, counts_row).reshape(T)

    counts = counts_row[0, :NUM_EXPERTS]
    g_pad = jnp.ceil(counts / BLK) * BLK
    ends = jnp.cumsum(g_pad)
    jblk = jnp.arange(NB, dtype=jnp.float32) * BLK
    bexp = jnp.minimum(
        jnp.sum((jblk[:, None] >= ends[None, :]).astype(jnp.int32), axis=1),
        NUM_EXPERTS - 1).astype(jnp.int32)

    scatter_k, gather_k = _sc_kernels()
    xs_i32 = lax.bitcast_convert_type(xs.reshape(T, H // 2, 2), jnp.int32)
    xsorted_i32 = scatter_k(xs_i32, slot)
    xsorted = lax.bitcast_convert_type(
        xsorted_i32, jnp.bfloat16).reshape(P, H)
    ys = _run_ffn(bexp, xsorted, gu_bf, dn_bf)
    ys_i32 = lax.bitcast_convert_type(ys.reshape(P, H // 2, 2), jnp.int32)
    ys_tok_i32 = gather_k(ys_i32, slot)
    ys_tok = lax.bitcast_convert_type(
        ys_tok_i32, jnp.bfloat16).reshape(T, H)
    return _run_shared(hs, ys_tok, sg_t, su_t, sd_t)


# speedup vs baseline: 2.1530x; 2.1530x over previous
import functools

import jax
import jax.numpy as jnp
from jax import lax
from jax.experimental import pallas as pl
from jax.experimental.pallas import tpu as pltpu
from jax.experimental.pallas import tpu_sc as plsc

NUM_EXPERTS = 8
HIDDEN = 1024
INTER = 2048
T = 4096
TCHUNK = 512
RCHUNKS = T // TCHUNK
BLK = 128
P = T + NUM_EXPERTS * BLK
NB = P // BLK
NC, NS = 2, 16
NW = NC * NS
TPW = T // NW
QTR = TPW // 4


def _router_body(hs_ref, rw_ref, xs_ref, idx_ref, rank_ref, counts_ref,
                 cnt_scr):
    t = pl.program_id(0)

    @pl.when(t == 0)
    def _():
        cnt_scr[...] = jnp.zeros((1, 128), jnp.float32)

    x32 = hs_ref[...]
    logits = jnp.dot(x32, rw_ref[...], preferred_element_type=jnp.float32)
    m = jnp.max(logits, axis=1, keepdims=True)
    s = jax.nn.sigmoid(m)
    iota_e = lax.broadcasted_iota(jnp.int32, logits.shape, 1)
    idx = jnp.min(jnp.where(logits == m, iota_e, NUM_EXPERTS), axis=1,
                  keepdims=True)
    onehot = (iota_e == idx).astype(jnp.float32)
    ri = lax.broadcasted_iota(jnp.int32, (TCHUNK, TCHUNK), 0)
    ci = lax.broadcasted_iota(jnp.int32, (TCHUNK, TCHUNK), 1)
    tril = (ci < ri).astype(jnp.float32)
    prev = jnp.dot(tril, onehot, preferred_element_type=jnp.float32)
    rank_local = jnp.sum(prev * onehot, axis=1, keepdims=True)
    cnt = cnt_scr[:, :NUM_EXPERTS]
    carry = jnp.sum(jnp.broadcast_to(cnt, onehot.shape) * onehot, axis=1,
                    keepdims=True)
    idx_ref[...] = idx
    rank_ref[...] = (rank_local + carry).astype(jnp.int32)
    xs_ref[...] = x32 * s
    cnt_scr[:, :NUM_EXPERTS] = cnt + jnp.sum(onehot, axis=0, keepdims=True)

    @pl.when(t == RCHUNKS - 1)
    def _():
        counts_ref[...] = cnt_scr[...]


def _run_router(hs, rw_t):
    return pl.pallas_call(
        _router_body,
        grid=(RCHUNKS,),
        in_specs=[
            pl.BlockSpec((TCHUNK, HIDDEN), lambda t: (t, 0)),
            pl.BlockSpec((HIDDEN, NUM_EXPERTS), lambda t: (0, 0)),
        ],
        out_specs=[
            pl.BlockSpec((TCHUNK, HIDDEN), lambda t: (t, 0)),
            pl.BlockSpec((TCHUNK, 1), lambda t: (t, 0)),
            pl.BlockSpec((TCHUNK, 1), lambda t: (t, 0)),
            pl.BlockSpec((1, 128), lambda t: (0, 0)),
        ],
        out_shape=[
            jax.ShapeDtypeStruct((T, HIDDEN), jnp.float32),
            jax.ShapeDtypeStruct((T, 1), jnp.int32),
            jax.ShapeDtypeStruct((T, 1), jnp.int32),
            jax.ShapeDtypeStruct((1, 128), jnp.float32),
        ],
        scratch_shapes=[pltpu.VMEM((1, 128), jnp.float32)],
    )(hs, rw_t)


def _slot_body(idx_ref, rank_ref, counts_ref, slot_ref):
    idx = idx_ref[...]
    cnt = counts_ref[:, :NUM_EXPERTS]
    g_pad = jnp.ceil(cnt / BLK) * BLK
    r8 = lax.broadcasted_iota(jnp.int32, (NUM_EXPERTS, NUM_EXPERTS), 0)
    c8 = lax.broadcasted_iota(jnp.int32, (NUM_EXPERTS, NUM_EXPERTS), 1)
    strict_upper = (r8 < c8).astype(jnp.float32)
    base = jnp.dot(g_pad, strict_upper, preferred_element_type=jnp.float32)
    iota_e = lax.broadcasted_iota(jnp.int32, (T, NUM_EXPERTS), 1)
    oh = (iota_e == idx).astype(jnp.float32)
    basetok = jnp.sum(jnp.broadcast_to(base, oh.shape) * oh, axis=1,
                      keepdims=True)
    slot_ref[...] = rank_ref[...] + basetok.astype(jnp.int32)


def _run_slot(idx2, rank2, counts_row):
    return pl.pallas_call(
        _slot_body,
        grid=(1,),
        in_specs=[
            pl.BlockSpec((T, 1), lambda i: (0, 0)),
            pl.BlockSpec((T, 1), lambda i: (0, 0)),
            pl.BlockSpec((1, 128), lambda i: (0, 0)),
        ],
        out_specs=pl.BlockSpec((T, 1), lambda i: (0, 0)),
        out_shape=jax.ShapeDtypeStruct((T, 1), jnp.int32),
    )(idx2, rank2, counts_row)


@functools.lru_cache(maxsize=None)
def _sc_kernels():
    mesh = plsc.VectorSubcoreMesh(core_axis_name="c", subcore_axis_name="s",
                                  num_cores=NC, num_subcores=NS)

    sc_scratch = [
        pltpu.VMEM((QTR,), jnp.int32),
        pltpu.VMEM((QTR,), jnp.int32),
        pltpu.VMEM((QTR,), jnp.int32),
        pltpu.VMEM((QTR,), jnp.int32),
        pltpu.VMEM((QTR, HIDDEN), jnp.float32),
        pltpu.VMEM((QTR, HIDDEN), jnp.float32),
        pltpu.SemaphoreType.DMA,
        pltpu.SemaphoreType.DMA,
        pltpu.SemaphoreType.DMA,
        pltpu.SemaphoreType.DMA,
    ]

    @functools.partial(
        pl.kernel,
        out_type=jax.ShapeDtypeStruct((P, HIDDEN), jnp.float32),
        mesh=mesh,
        scratch_types=sc_scratch,
    )
    def _scatter_kernel(xs_hbm, slot_hbm, xsorted_hbm, s0, s1, s2, s3,
                        buf_a, buf_b, sem_la, sem_lb, sem_sa, sem_sb):
        wid = lax.axis_index("s") * NC + lax.axis_index("c")
        t0 = wid * TPW
        srefs = (s0, s1, s2, s3)
        for q in range(4):
            pltpu.sync_copy(slot_hbm.at[pl.ds(t0 + q * QTR, QTR)], srefs[q])
        l0 = pltpu.async_copy(xs_hbm.at[pl.ds(t0, QTR)], buf_a, sem_la)
        l1 = pltpu.async_copy(xs_hbm.at[pl.ds(t0 + QTR, QTR)], buf_b, sem_lb)
        l0.wait()
        w0 = pltpu.async_copy(buf_a, xsorted_hbm.at[s0], sem_sa)
        l1.wait()
        w1 = pltpu.async_copy(buf_b, xsorted_hbm.at[s1], sem_sb)
        w0.wait()
        l2 = pltpu.async_copy(xs_hbm.at[pl.ds(t0 + 2 * QTR, QTR)], buf_a,
                              sem_la)
        w1.wait()
        l3 = pltpu.async_copy(xs_hbm.at[pl.ds(t0 + 3 * QTR, QTR)], buf_b,
                              sem_lb)
        l2.wait()
        w2 = pltpu.async_copy(buf_a, xsorted_hbm.at[s2], sem_sa)
        l3.wait()
        w3 = pltpu.async_copy(buf_b, xsorted_hbm.at[s3], sem_sb)
        w2.wait()
        w3.wait()

    @functools.partial(
        pl.kernel,
        out_type=jax.ShapeDtypeStruct((T, HIDDEN), jnp.float32),
        mesh=mesh,
        scratch_types=sc_scratch,
    )
    def _gather_kernel(ys_hbm, slot_hbm, out_hbm, s0, s1, s2, s3,
                       buf_a, buf_b, sem_la, sem_lb, sem_sa, sem_sb):
        wid = lax.axis_index("s") * NC + lax.axis_index("c")
        t0 = wid * TPW
        srefs = (s0, s1, s2, s3)
        for q in range(4):
            pltpu.sync_copy(slot_hbm.at[pl.ds(t0 + q * QTR, QTR)], srefs[q])
        g0 = pltpu.async_copy(ys_hbm.at[s0], buf_a, sem_la)
        g1 = pltpu.async_copy(ys_hbm.at[s1], buf_b, sem_lb)
        g0.wait()
        w0 = pltpu.async_copy(buf_a, out_hbm.at[pl.ds(t0, QTR)], sem_sa)
        g1.wait()
        w1 = pltpu.async_copy(buf_b, out_hbm.at[pl.ds(t0 + QTR, QTR)],
                              sem_sb)
        w0.wait()
        g2 = pltpu.async_copy(ys_hbm.at[s2], buf_a, sem_la)
        w1.wait()
        g3 = pltpu.async_copy(ys_hbm.at[s3], buf_b, sem_lb)
        g2.wait()
        w2 = pltpu.async_copy(buf_a, out_hbm.at[pl.ds(t0 + 2 * QTR, QTR)],
                              sem_sa)
        g3.wait()
        w3 = pltpu.async_copy(buf_b, out_hbm.at[pl.ds(t0 + 3 * QTR, QTR)],
                              sem_sb)
        w2.wait()
        w3.wait()

    return _scatter_kernel, _gather_kernel


def _ffn_body(be_ref, xs_ref, gu_ref, dn_ref, out_ref):
    x = xs_ref[...].astype(jnp.bfloat16)
    gu = jnp.dot(x, gu_ref[0], preferred_element_type=jnp.float32)
    gate = gu[:, :INTER]
    up = gu[:, INTER:]
    gated = (up * (gate * jax.nn.sigmoid(gate))).astype(jnp.bfloat16)
    out_ref[...] = jnp.dot(gated, dn_ref[0],
                           preferred_element_type=jnp.float32)


def _run_ffn(bexp, xsorted, gu_bf, dn_bf):
    grid_spec = pltpu.PrefetchScalarGridSpec(
        num_scalar_prefetch=1,
        grid=(NB,),
        in_specs=[
            pl.BlockSpec((BLK, HIDDEN), lambda j, be: (j, 0)),
            pl.BlockSpec((1, HIDDEN, 2 * INTER), lambda j, be: (be[j], 0, 0)),
            pl.BlockSpec((1, INTER, HIDDEN), lambda j, be: (be[j], 0, 0)),
        ],
        out_specs=pl.BlockSpec((BLK, HIDDEN), lambda j, be: (j, 0)),
    )
    return pl.pallas_call(
        _ffn_body,
        grid_spec=grid_spec,
        out_shape=jax.ShapeDtypeStruct((P, HIDDEN), jnp.float32),
    )(bexp, xsorted, gu_bf, dn_bf)


def _shared_body(hs_ref, yt_ref, sg_ref, su_ref, sd_ref, out_ref):
    x = hs_ref[...].astype(jnp.bfloat16)
    g = jnp.dot(x, sg_ref[...], preferred_element_type=jnp.float32)
    u = jnp.dot(x, su_ref[...], preferred_element_type=jnp.float32)
    act = (u * (g * jax.nn.sigmoid(g))).astype(jnp.bfloat16)
    sh = jnp.dot(act, sd_ref[...], preferred_element_type=jnp.float32)
    out_ref[...] = sh + yt_ref[...]


def _run_shared(hs, ys_tok, sg_t, su_t, sd_t):
    return pl.pallas_call(
        _shared_body,
        grid=(RCHUNKS,),
        in_specs=[
            pl.BlockSpec((TCHUNK, HIDDEN), lambda t: (t, 0)),
            pl.BlockSpec((TCHUNK, HIDDEN), lambda t: (t, 0)),
            pl.BlockSpec((HIDDEN, INTER), lambda t: (0, 0)),
            pl.BlockSpec((HIDDEN, INTER), lambda t: (0, 0)),
            pl.BlockSpec((INTER, HIDDEN), lambda t: (0, 0)),
        ],
        out_specs=pl.BlockSpec((TCHUNK, HIDDEN), lambda t: (t, 0)),
        out_shape=jax.ShapeDtypeStruct((T, HIDDEN), jnp.float32),
    )(hs, ys_tok, sg_t, su_t, sd_t)


def kernel(hidden_states, gate_up_proj, down_proj, router_w, shared_gate_w,
           shared_up_w, shared_down_w):
    B, S, H = hidden_states.shape
    hs = hidden_states.reshape(B * S, H)
    rw_t = router_w.T.astype(jnp.float32)
    gu_bf = gate_up_proj.astype(jnp.bfloat16)
    dn_bf = down_proj.astype(jnp.bfloat16)
    sg_t = shared_gate_w.T.astype(jnp.bfloat16)
    su_t = shared_up_w.T.astype(jnp.bfloat16)
    sd_t = shared_down_w.T.astype(jnp.bfloat16)

    xs, idx2, rank2, counts_row = _run_router(hs, rw_t)
    slot = _run_slot(idx2, rank2, counts_row).reshape(T)

    counts = counts_row[0, :NUM_EXPERTS]
    g_pad = jnp.ceil(counts / BLK) * BLK
    ends = jnp.cumsum(g_pad)
    jblk = jnp.arange(NB, dtype=jnp.float32) * BLK
    bexp = jnp.minimum(
        jnp.sum((jblk[:, None] >= ends[None, :]).astype(jnp.int32), axis=1),
        NUM_EXPERTS - 1).astype(jnp.int32)

    scatter_k, gather_k = _sc_kernels()
    xsorted = scatter_k(xs, slot)
    ys = _run_ffn(bexp, xsorted, gu_bf, dn_bf)
    ys_tok = gather_k(ys, slot)
    return _run_shared(hs, ys_tok, sg_t, su_t, sd_t)


# speedup vs baseline: 2.3504x; 1.0917x over previous
import functools

import jax
import jax.numpy as jnp
from jax import lax
from jax.experimental import pallas as pl
from jax.experimental.pallas import tpu as pltpu
from jax.experimental.pallas import tpu_sc as plsc

NUM_EXPERTS = 8
HIDDEN = 1024
INTER = 2048
T = 4096
TCHUNK = 512
RCHUNKS = T // TCHUNK
BLK = 128
P = T + NUM_EXPERTS * BLK
NB = P // BLK
NC, NS = 2, 16
NW = NC * NS
TPW = T // NW
QTR = TPW // 4


def _router_body(hs_ref, rw_ref, xs_ref, idx_ref, rank_ref, counts_ref,
                 cnt_scr):
    t = pl.program_id(0)

    @pl.when(t == 0)
    def _():
        cnt_scr[...] = jnp.zeros((1, 128), jnp.float32)

    x32 = hs_ref[...]
    logits = jnp.dot(x32, rw_ref[...], preferred_element_type=jnp.float32)
    m = jnp.max(logits, axis=1, keepdims=True)
    s = jax.nn.sigmoid(m)
    iota_e = lax.broadcasted_iota(jnp.int32, logits.shape, 1)
    idx = jnp.min(jnp.where(logits == m, iota_e, NUM_EXPERTS), axis=1,
                  keepdims=True)
    onehot = (iota_e == idx).astype(jnp.float32)
    ri = lax.broadcasted_iota(jnp.int32, (TCHUNK, TCHUNK), 0)
    ci = lax.broadcasted_iota(jnp.int32, (TCHUNK, TCHUNK), 1)
    tril = (ci < ri).astype(jnp.float32)
    prev = jnp.dot(tril, onehot, preferred_element_type=jnp.float32)
    rank_local = jnp.sum(prev * onehot, axis=1, keepdims=True)
    cnt = cnt_scr[:, :NUM_EXPERTS]
    carry = jnp.sum(jnp.broadcast_to(cnt, onehot.shape) * onehot, axis=1,
                    keepdims=True)
    idx_ref[...] = idx
    rank_ref[...] = (rank_local + carry).astype(jnp.int32)
    xs_ref[...] = x32 * s
    cnt_scr[:, :NUM_EXPERTS] = cnt + jnp.sum(onehot, axis=0, keepdims=True)

    @pl.when(t == RCHUNKS - 1)
    def _():
        counts_ref[...] = cnt_scr[...]


def _run_router(hs, rw_t):
    return pl.pallas_call(
        _router_body,
        grid=(RCHUNKS,),
        in_specs=[
            pl.BlockSpec((TCHUNK, HIDDEN), lambda t: (t, 0)),
            pl.BlockSpec((HIDDEN, NUM_EXPERTS), lambda t: (0, 0)),
        ],
        out_specs=[
            pl.BlockSpec((TCHUNK, HIDDEN), lambda t: (t, 0)),
            pl.BlockSpec((TCHUNK, 1), lambda t: (t, 0)),
            pl.BlockSpec((TCHUNK, 1), lambda t: (t, 0)),
            pl.BlockSpec((1, 128), lambda t: (0, 0)),
        ],
        out_shape=[
            jax.ShapeDtypeStruct((T, HIDDEN), jnp.float32),
            jax.ShapeDtypeStruct((T, 1), jnp.int32),
            jax.ShapeDtypeStruct((T, 1), jnp.int32),
            jax.ShapeDtypeStruct((1, 128), jnp.float32),
        ],
        scratch_shapes=[pltpu.VMEM((1, 128), jnp.float32)],
    )(hs, rw_t)


def _slot_body(idx_ref, rank_ref, counts_ref, slot_ref):
    idx = idx_ref[...]
    cnt = counts_ref[:, :NUM_EXPERTS]
    g_pad = jnp.ceil(cnt / BLK) * BLK
    r8 = lax.broadcasted_iota(jnp.int32, (NUM_EXPERTS, NUM_EXPERTS), 0)
    c8 = lax.broadcasted_iota(jnp.int32, (NUM_EXPERTS, NUM_EXPERTS), 1)
    strict_upper = (r8 < c8).astype(jnp.float32)
    base = jnp.dot(g_pad, strict_upper, preferred_element_type=jnp.float32)
    iota_e = lax.broadcasted_iota(jnp.int32, (T, NUM_EXPERTS), 1)
    oh = (iota_e == idx).astype(jnp.float32)
    basetok = jnp.sum(jnp.broadcast_to(base, oh.shape) * oh, axis=1,
                      keepdims=True)
    slot_ref[...] = rank_ref[...] + basetok.astype(jnp.int32)


def _run_slot(idx2, rank2, counts_row):
    return pl.pallas_call(
        _slot_body,
        grid=(1,),
        in_specs=[
            pl.BlockSpec((T, 1), lambda i: (0, 0)),
            pl.BlockSpec((T, 1), lambda i: (0, 0)),
            pl.BlockSpec((1, 128), lambda i: (0, 0)),
        ],
        out_specs=pl.BlockSpec((T, 1), lambda i: (0, 0)),
        out_shape=jax.ShapeDtypeStruct((T, 1), jnp.int32),
    )(idx2, rank2, counts_row)


@functools.lru_cache(maxsize=None)
def _sc_kernels():
    mesh = plsc.VectorSubcoreMesh(core_axis_name="c", subcore_axis_name="s",
                                  num_cores=NC, num_subcores=NS)

    sc_scratch = [
        pltpu.VMEM((QTR,), jnp.int32),
        pltpu.VMEM((QTR,), jnp.int32),
        pltpu.VMEM((QTR,), jnp.int32),
        pltpu.VMEM((QTR,), jnp.int32),
        pltpu.VMEM((QTR, HIDDEN), jnp.float32),
        pltpu.VMEM((QTR, HIDDEN), jnp.float32),
        pltpu.SemaphoreType.DMA,
        pltpu.SemaphoreType.DMA,
        pltpu.SemaphoreType.DMA,
        pltpu.SemaphoreType.DMA,
    ]

    @functools.partial(
        pl.kernel,
        out_type=jax.ShapeDtypeStruct((P, HIDDEN), jnp.float32),
        mesh=mesh,
        scratch_types=sc_scratch,
    )
    def _scatter_kernel(xs_hbm, slot_hbm, xsorted_hbm, s0, s1, s2, s3,
                        buf_a, buf_b, sem_la, sem_lb, sem_sa, sem_sb):
        wid = lax.axis_index("s") * NC + lax.axis_index("c")
        t0 = wid * TPW
        srefs = (s0, s1, s2, s3)
        for q in range(4):
            pltpu.sync_copy(slot_hbm.at[pl.ds(t0 + q * QTR, QTR)], srefs[q])
        l0 = pltpu.async_copy(xs_hbm.at[pl.ds(t0, QTR)], buf_a, sem_la)
        l1 = pltpu.async_copy(xs_hbm.at[pl.ds(t0 + QTR, QTR)], buf_b, sem_lb)
        l0.wait()
        w0 = pltpu.async_copy(buf_a, xsorted_hbm.at[s0], sem_sa)
        l1.wait()
        w1 = pltpu.async_copy(buf_b, xsorted_hbm.at[s1], sem_sb)
        w0.wait()
        l2 = pltpu.async_copy(xs_hbm.at[pl.ds(t0 + 2 * QTR, QTR)], buf_a,
                              sem_la)
        w1.wait()
        l3 = pltpu.async_copy(xs_hbm.at[pl.ds(t0 + 3 * QTR, QTR)], buf_b,
                              sem_lb)
        l2.wait()
        w2 = pltpu.async_copy(buf_a, xsorted_hbm.at[s2], sem_sa)
        l3.wait()
        w3 = pltpu.async_copy(buf_b, xsorted_hbm.at[s3], sem_sb)
        w2.wait()
        w3.wait()

    @functools.partial(
        pl.kernel,
        out_type=jax.ShapeDtypeStruct((T, HIDDEN), jnp.float32),
        mesh=mesh,
        scratch_types=sc_scratch,
    )
    def _gather_kernel(ys_hbm, slot_hbm, out_hbm, s0, s1, s2, s3,
                       buf_a, buf_b, sem_la, sem_lb, sem_sa, sem_sb):
        wid = lax.axis_index("s") * NC + lax.axis_index("c")
        t0 = wid * TPW
        srefs = (s0, s1, s2, s3)
        for q in range(4):
            pltpu.sync_copy(slot_hbm.at[pl.ds(t0 + q * QTR, QTR)], srefs[q])
        g0 = pltpu.async_copy(ys_hbm.at[s0], buf_a, sem_la)
        g1 = pltpu.async_copy(ys_hbm.at[s1], buf_b, sem_lb)
        g0.wait()
        w0 = pltpu.async_copy(buf_a, out_hbm.at[pl.ds(t0, QTR)], sem_sa)
        g1.wait()
        w1 = pltpu.async_copy(buf_b, out_hbm.at[pl.ds(t0 + QTR, QTR)],
                              sem_sb)
        w0.wait()
        g2 = pltpu.async_copy(ys_hbm.at[s2], buf_a, sem_la)
        w1.wait()
        g3 = pltpu.async_copy(ys_hbm.at[s3], buf_b, sem_lb)
        g2.wait()
        w2 = pltpu.async_copy(buf_a, out_hbm.at[pl.ds(t0 + 2 * QTR, QTR)],
                              sem_sa)
        g3.wait()
        w3 = pltpu.async_copy(buf_b, out_hbm.at[pl.ds(t0 + 3 * QTR, QTR)],
                              sem_sb)
        w2.wait()
        w3.wait()

    return _scatter_kernel, _gather_kernel


def _expert_changed(be_ref):
    j = pl.program_id(0)
    prev = be_ref[jnp.maximum(j - 1, 0)]
    return jnp.logical_or(j == 0, be_ref[j] != prev)


def _ffn1_body(be_ref, xs_ref, gu_ref, out_ref, wbf):
    @pl.when(_expert_changed(be_ref))
    def _():
        wbf[...] = gu_ref[0].astype(jnp.bfloat16)

    x = xs_ref[...].astype(jnp.bfloat16)
    gu = jnp.dot(x, wbf[...], preferred_element_type=jnp.float32)
    gate = gu[:, :INTER]
    up = gu[:, INTER:]
    out_ref[...] = (up * (gate * jax.nn.sigmoid(gate))).astype(jnp.bfloat16)


def _ffn2_body(be_ref, gated_ref, dn_ref, out_ref, dbf):
    @pl.when(_expert_changed(be_ref))
    def _():
        dbf[...] = dn_ref[0].astype(jnp.bfloat16)

    out_ref[...] = jnp.dot(gated_ref[...], dbf[...],
                           preferred_element_type=jnp.float32)


def _run_ffn(bexp, xsorted, gate_up_proj, down_proj):
    gated = pl.pallas_call(
        _ffn1_body,
        grid_spec=pltpu.PrefetchScalarGridSpec(
            num_scalar_prefetch=1,
            grid=(NB,),
            in_specs=[
                pl.BlockSpec((BLK, HIDDEN), lambda j, be: (j, 0)),
                pl.BlockSpec((1, HIDDEN, 2 * INTER),
                             lambda j, be: (be[j], 0, 0)),
            ],
            out_specs=pl.BlockSpec((BLK, INTER), lambda j, be: (j, 0)),
            scratch_shapes=[pltpu.VMEM((HIDDEN, 2 * INTER), jnp.bfloat16)],
        ),
        out_shape=jax.ShapeDtypeStruct((P, INTER), jnp.bfloat16),
    )(bexp, xsorted, gate_up_proj)
    return pl.pallas_call(
        _ffn2_body,
        grid_spec=pltpu.PrefetchScalarGridSpec(
            num_scalar_prefetch=1,
            grid=(NB,),
            in_specs=[
                pl.BlockSpec((BLK, INTER), lambda j, be: (j, 0)),
                pl.BlockSpec((1, INTER, HIDDEN),
                             lambda j, be: (be[j], 0, 0)),
            ],
            out_specs=pl.BlockSpec((BLK, HIDDEN), lambda j, be: (j, 0)),
            scratch_shapes=[pltpu.VMEM((INTER, HIDDEN), jnp.bfloat16)],
        ),
        out_shape=jax.ShapeDtypeStruct((P, HIDDEN), jnp.float32),
    )(bexp, gated, down_proj)


_DN_T = (((1,), (1,)), ((), ()))


def _shared_body(hs_ref, yt_ref, sg_ref, su_ref, sd_ref, out_ref,
                 sgbf, subf, sdbf):
    @pl.when(pl.program_id(0) == 0)
    def _():
        sgbf[...] = sg_ref[...].astype(jnp.bfloat16)
        subf[...] = su_ref[...].astype(jnp.bfloat16)
        sdbf[...] = sd_ref[...].astype(jnp.bfloat16)

    x = hs_ref[...].astype(jnp.bfloat16)
    g = lax.dot_general(x, sgbf[...], _DN_T,
                        preferred_element_type=jnp.float32)
    u = lax.dot_general(x, subf[...], _DN_T,
                        preferred_element_type=jnp.float32)
    act = (u * (g * jax.nn.sigmoid(g))).astype(jnp.bfloat16)
    sh = lax.dot_general(act, sdbf[...], _DN_T,
                         preferred_element_type=jnp.float32)
    out_ref[...] = sh + yt_ref[...]


def _run_shared(hs, ys_tok, shared_gate_w, shared_up_w, shared_down_w):
    return pl.pallas_call(
        _shared_body,
        grid=(RCHUNKS,),
        in_specs=[
            pl.BlockSpec((TCHUNK, HIDDEN), lambda t: (t, 0)),
            pl.BlockSpec((TCHUNK, HIDDEN), lambda t: (t, 0)),
            pl.BlockSpec((INTER, HIDDEN), lambda t: (0, 0)),
            pl.BlockSpec((INTER, HIDDEN), lambda t: (0, 0)),
            pl.BlockSpec((HIDDEN, INTER), lambda t: (0, 0)),
        ],
        out_specs=pl.BlockSpec((TCHUNK, HIDDEN), lambda t: (t, 0)),
        out_shape=jax.ShapeDtypeStruct((T, HIDDEN), jnp.float32),
        scratch_shapes=[
            pltpu.VMEM((INTER, HIDDEN), jnp.bfloat16),
            pltpu.VMEM((INTER, HIDDEN), jnp.bfloat16),
            pltpu.VMEM((HIDDEN, INTER), jnp.bfloat16),
        ],
    )(hs, ys_tok, shared_gate_w, shared_up_w, shared_down_w)


def kernel(hidden_states, gate_up_proj, down_proj, router_w, shared_gate_w,
           shared_up_w, shared_down_w):
    B, S, H = hidden_states.shape
    hs = hidden_states.reshape(B * S, H)
    rw_t = router_w.T.astype(jnp.float32)

    xs, idx2, rank2, counts_row = _run_router(hs, rw_t)
    slot = _run_slot(idx2, rank2, counts_row).reshape(T)

    counts = counts_row[0, :NUM_EXPERTS]
    g_pad = jnp.ceil(counts / BLK) * BLK
    ends = jnp.cumsum(g_pad)
    jblk = jnp.arange(NB, dtype=jnp.float32) * BLK
    bexp = jnp.minimum(
        jnp.sum((jblk[:, None] >= ends[None, :]).astype(jnp.int32), axis=1),
        NUM_EXPERTS - 1).astype(jnp.int32)

    scatter_k, gather_k = _sc_kernels()
    xsorted = scatter_k(xs, slot)
    ys = _run_ffn(bexp, xsorted, gate_up_proj, down_proj)
    ys_tok = gather_k(ys, slot)
    return _run_shared(hs, ys_tok, shared_gate_w, shared_up_w, shared_down_w)
